# baseline (speedup 1.0000x reference)
import jax
import jax.numpy as jnp
from jax.experimental import pallas as pl


def _add_body(x_ref, pos_ref, o_ref):
    o_ref[...] = x_ref[...] + pos_ref[...]


def kernel(patch, pos_table):
    B, P, D = patch.shape
    PD = P * D
    x = patch.reshape(B, PD)
    pos = pos_table.reshape(1, PD)
    BB = 8
    CC = 12288
    out = pl.pallas_call(
        _add_body,
        grid=(B // BB, PD // CC),
        in_specs=[
            pl.BlockSpec((BB, CC), lambda i, j: (i, j)),
            pl.BlockSpec((1, CC), lambda i, j: (0, j)),
        ],
        out_specs=pl.BlockSpec((BB, CC), lambda i, j: (i, j)),
        out_shape=jax.ShapeDtypeStruct((B, PD), jnp.float32),
    )(x, pos)
    return out.reshape(B, P, D)


# speedup vs baseline: 1.8780x; 1.8780x over previous
import jax
import jax.numpy as jnp
from jax.experimental import pallas as pl


def _add_body(x_ref, pos_ref, o_ref):
    o_ref[...] = x_ref[...] + pos_ref[...][None]


def kernel(patch, pos_table):
    B, P, D = patch.shape
    BB = 8
    PP = 512
    out = pl.pallas_call(
        _add_body,
        grid=(B // BB, P // PP),
        in_specs=[
            pl.BlockSpec((BB, PP, D), lambda i, j: (i, j, 0)),
            pl.BlockSpec((PP, D), lambda i, j: (j, 0)),
        ],
        out_specs=pl.BlockSpec((BB, PP, D), lambda i, j: (i, j, 0)),
        out_shape=jax.ShapeDtypeStruct((B, P, D), jnp.float32),
    )(patch, pos_table)
    return out


# speedup vs baseline: 8.9306x; 4.7554x over previous
import jax
import jax.numpy as jnp
from jax.experimental import pallas as pl


def _add_body(x_ref, pos_ref, o_ref):
    o_ref[...] = x_ref[...] + pos_ref[...][None]


def kernel(patch, pos_table):
    B, P, D = patch.shape
    xt = jnp.transpose(patch, (0, 2, 1))
    post = jnp.transpose(pos_table, (1, 0))
    BB = 8
    out_t = pl.pallas_call(
        _add_body,
        grid=(B // BB,),
        in_specs=[
            pl.BlockSpec((BB, D, P), lambda i: (i, 0, 0)),
            pl.BlockSpec((D, P), lambda i: (0, 0)),
        ],
        out_specs=pl.BlockSpec((BB, D, P), lambda i: (i, 0, 0)),
        out_shape=jax.ShapeDtypeStruct((B, D, P), jnp.float32),
    )(xt, post)
    return jnp.transpose(out_t, (0, 2, 1))
